# Initial kernel scaffold; baseline (speedup 1.0000x reference)
import jax
import jax.numpy as jnp
from jax.experimental import pallas as pl


def kernel(rois, mrcnn_class, mrcnn_bbox):
    raise NotImplementedError("write your pallas kernel here")


# speedup vs baseline: 5.0723x; 5.0723x over previous
import functools

import jax
import jax.numpy as jnp
from jax.experimental import pallas as pl
from jax.experimental.pallas import tpu as pltpu

B = 2
N = 5000
C = 81
L = 128
R = 40
NP = R * L
MAX_OUT = 100
MIN_CONF = 0.05
NMS_THR = 0.3
STD = (0.1, 0.1, 0.2, 0.2)
NEG = -jnp.inf


def _body(probs_ref, d0_ref, d1_ref, d2_ref, d3_ref, rois_ref, out_ref,
          scores_ref, oy1_ref, ox1_ref, oy2_ref, ox2_ref,
          ry1_ref, rx1_ref, ry2_ref, rx2_ref, area_ref, cls_ref):
    def cls_step(c, carry):
        m, cid, e0, e1, e2, e3 = carry
        p = probs_ref[0, c]
        upd = p > m
        m = jnp.where(upd, p, m)
        cid = jnp.where(upd, c, cid)
        e0 = jnp.where(upd, d0_ref[0, c], e0)
        e1 = jnp.where(upd, d1_ref[0, c], e1)
        e2 = jnp.where(upd, d2_ref[0, c], e2)
        e3 = jnp.where(upd, d3_ref[0, c], e3)
        return m, cid, e0, e1, e2, e3

    init = (probs_ref[0, 0], jnp.zeros((R, L), jnp.int32),
            d0_ref[0, 0], d1_ref[0, 0], d2_ref[0, 0], d3_ref[0, 0])
    m, cid, e0, e1, e2, e3 = jax.lax.fori_loop(1, C, cls_step, init)

    y1 = rois_ref[0, 0]
    x1 = rois_ref[0, 1]
    y2 = rois_ref[0, 2]
    x2 = rois_ref[0, 3]
    h = y2 - y1
    w = x2 - x1
    cy = y1 + 0.5 * h
    cx = x1 + 0.5 * w
    cy = cy + (e0 * STD[0]) * h
    cx = cx + (e1 * STD[1]) * w
    h = h * jnp.exp(e2 * STD[2])
    w = w * jnp.exp(e3 * STD[3])
    ny1 = cy - 0.5 * h
    nx1 = cx - 0.5 * w
    ny2 = ny1 + h
    nx2 = nx1 + w
    ry1 = jnp.clip(ny1, 0.0, 1.0)
    rx1 = jnp.clip(nx1, 0.0, 1.0)
    ry2 = jnp.clip(ny2, 0.0, 1.0)
    rx2 = jnp.clip(nx2, 0.0, 1.0)

    clsf = cid.astype(jnp.float32)
    off = clsf * 4.0
    oy1 = ry1 + off
    ox1 = rx1 + off
    oy2 = ry2 + off
    ox2 = rx2 + off

    iota_r = jax.lax.broadcasted_iota(jnp.int32, (R, L), 0)
    iota_l = jax.lax.broadcasted_iota(jnp.int32, (R, L), 1)
    flat = iota_r * L + iota_l
    valid0 = (cid > 0) & (m >= MIN_CONF) & (flat < N)
    scores_ref[...] = jnp.where(valid0, m, NEG)
    oy1_ref[...] = oy1
    ox1_ref[...] = ox1
    oy2_ref[...] = oy2
    ox2_ref[...] = ox2
    ry1_ref[...] = ry1
    rx1_ref[...] = rx1
    ry2_ref[...] = ry2
    rx2_ref[...] = rx2
    area_ref[...] = (oy2 - oy1) * (ox2 - ox1)
    cls_ref[...] = clsf

    lane8 = jax.lax.broadcasted_iota(jnp.int32, (1, L), 1)

    def _extract(ref, si, li):
        row = ref[pl.ds(si, 1), :]
        return jnp.sum(jnp.where(lane8 == li, row, 0.0))

    def nms_step(step, _):
        s = scores_ref[...]
        mx = jnp.max(s)
        idx = jnp.min(jnp.where(s == mx, flat, NP))
        si = idx // L
        li = idx % L
        b_y1 = _extract(oy1_ref, si, li)
        b_x1 = _extract(ox1_ref, si, li)
        b_y2 = _extract(oy2_ref, si, li)
        b_x2 = _extract(ox2_ref, si, li)
        yy1 = jnp.maximum(b_y1, oy1_ref[...])
        xx1 = jnp.maximum(b_x1, ox1_ref[...])
        yy2 = jnp.minimum(b_y2, oy2_ref[...])
        xx2 = jnp.minimum(b_x2, ox2_ref[...])
        inter = jnp.maximum(yy2 - yy1, 0.0) * jnp.maximum(xx2 - xx1, 0.0)
        a1 = (b_y2 - b_y1) * (b_x2 - b_x1)
        iou = inter / (a1 + area_ref[...] - inter + 1e-9)
        suppress = (iou > NMS_THR) | (flat == idx)
        scores_ref[...] = jnp.where(suppress, NEG, s)

        valid = mx > NEG
        v0 = _extract(ry1_ref, si, li)
        v1 = _extract(rx1_ref, si, li)
        v2 = _extract(ry2_ref, si, li)
        v3 = _extract(rx2_ref, si, li)
        v4 = _extract(cls_ref, si, li)
        row = jnp.where(lane8 == 0, v0,
              jnp.where(lane8 == 1, v1,
              jnp.where(lane8 == 2, v2,
              jnp.where(lane8 == 3, v3,
              jnp.where(lane8 == 4, v4,
              jnp.where(lane8 == 5, mx, 0.0))))))
        out_ref[0, pl.ds(step, 1)] = jnp.where(valid, row, 0.0)
        return 0

    jax.lax.fori_loop(0, MAX_OUT, nms_step, 0)


@jax.jit
def kernel(rois, mrcnn_class, mrcnn_bbox):
    probs_t = jnp.pad(mrcnn_class.transpose(0, 2, 1),
                      ((0, 0), (0, 0), (0, NP - N))).reshape(B, C, R, L)
    d = jnp.pad(mrcnn_bbox.transpose(0, 2, 3, 1),
                ((0, 0), (0, 0), (0, 0), (0, NP - N))).reshape(B, C, 4, R, L)
    d0 = d[:, :, 0]
    d1 = d[:, :, 1]
    d2 = d[:, :, 2]
    d3 = d[:, :, 3]
    rois_t = jnp.pad(rois.transpose(0, 2, 1),
                     ((0, 0), (0, 0), (0, NP - N))).reshape(B, 4, R, L)

    in_specs = [
        pl.BlockSpec((1, C, R, L), lambda b: (b, 0, 0, 0)),
        pl.BlockSpec((1, C, R, L), lambda b: (b, 0, 0, 0)),
        pl.BlockSpec((1, C, R, L), lambda b: (b, 0, 0, 0)),
        pl.BlockSpec((1, C, R, L), lambda b: (b, 0, 0, 0)),
        pl.BlockSpec((1, C, R, L), lambda b: (b, 0, 0, 0)),
        pl.BlockSpec((1, 4, R, L), lambda b: (b, 0, 0, 0)),
    ]
    out = pl.pallas_call(
        _body,
        grid=(B,),
        in_specs=in_specs,
        out_specs=pl.BlockSpec((1, MAX_OUT, L), lambda b: (b, 0, 0)),
        out_shape=jax.ShapeDtypeStruct((B, MAX_OUT, L), jnp.float32),
        scratch_shapes=[pltpu.VMEM((R, L), jnp.float32)] * 11,
    )(probs_t, d0, d1, d2, d3, rois_t)
    return out[:, :, :6]
